# Initial kernel scaffold; baseline (speedup 1.0000x reference)
import jax
import jax.numpy as jnp
from jax.experimental import pallas as pl


def kernel(x, edge_attr, gin_eps, gin_W1, gin_b1, gin_g1, gin_be1, gin_W2, gin_b2, bn_g, bn_b, vn_emb0, vn_W1, vn_b1, vn_g1, vn_be1, vn_W2, vn_b2, vn_g2, vn_be2, mlp_W, mlp_b, edge_index, batch):
    raise NotImplementedError("write your pallas kernel here")


# speedup vs baseline: 3.3705x; 3.3705x over previous
import jax
import jax.numpy as jnp
from jax import lax
from jax.experimental import pallas as pl
from jax.experimental.pallas import tpu as pltpu
from jax.experimental.pallas import tpu_sc as plsc

N = 10000
E = 320000
D = 128
HID = 256
L = 5
G = 128
NC = 1

RB = 2000
NBLK = N // RB

NCORE = 2
NSUB = 16
NW = NCORE * NSUB
EPW = E // NW
CK = 80
NCH = EPW // CK
CPR = 632
LASTR = N - (NSUB - 1) * CPR


def _edge_body(hl, ea, src, dst, zer, out, sidx, didx, gbuf, ebuf, agg_sh,
               sem_g, sem_e):
    cid = lax.axis_index("c")
    sid = lax.axis_index("s")
    wid = cid * NSUB + sid

    @pl.when(sid < NSUB - 1)
    def _():
        pltpu.sync_copy(zer, agg_sh.at[pl.ds(sid * CPR, CPR)])

    @pl.when(sid == NSUB - 1)
    def _():
        pltpu.sync_copy(zer.at[pl.ds(0, LASTR)],
                        agg_sh.at[pl.ds((NSUB - 1) * CPR, LASTR)])

    plsc.subcore_barrier()

    def chunk(c, carry):
        base = pl.multiple_of(wid * EPW + c * CK, 8)
        pltpu.sync_copy(src.at[pl.ds(base, CK)], sidx)
        pltpu.sync_copy(dst.at[pl.ds(base, CK)], didx)
        cp_g = pltpu.async_copy(hl.at[sidx], gbuf, sem_g)
        cp_e = pltpu.async_copy(ea.at[pl.ds(base, CK)], ebuf, sem_e)
        cp_g.wait()
        cp_e.wait()

        def row(i, c2):
            for j in range(D // 16):
                s = pl.ds(j * 16, 16)
                gbuf[i, s] = jnp.maximum(gbuf[i, s] + ebuf[i, s], 0.0)
            return c2

        lax.fori_loop(0, CK, row, 0)
        pltpu.sync_copy(gbuf, agg_sh.at[didx], add=True)
        return carry

    lax.fori_loop(0, NCH, chunk, 0)
    plsc.subcore_barrier()

    @pl.when(sid < NSUB - 1)
    def _():
        pltpu.sync_copy(agg_sh.at[pl.ds(sid * CPR, CPR)],
                        out.at[cid, pl.ds(sid * CPR, CPR)])

    @pl.when(sid == NSUB - 1)
    def _():
        pltpu.sync_copy(agg_sh.at[pl.ds((NSUB - 1) * CPR, LASTR)],
                        out.at[cid, pl.ds((NSUB - 1) * CPR, LASTR)])


def _sc_edge(hl, ea, src, dst, zer):
    kern = pl.kernel(
        _edge_body,
        out_type=jax.ShapeDtypeStruct((NCORE, N, D), jnp.float32),
        mesh=plsc.VectorSubcoreMesh(core_axis_name="c", subcore_axis_name="s"),
        scratch_types=[
            pltpu.VMEM((CK,), jnp.int32),
            pltpu.VMEM((CK,), jnp.int32),
            pltpu.VMEM((CK, D), jnp.float32),
            pltpu.VMEM((CK, D), jnp.float32),
            pltpu.VMEM_SHARED((N, D), jnp.float32),
            pltpu.SemaphoreType.DMA,
            pltpu.SemaphoreType.DMA,
        ],
    )
    return kern(hl, ea, src, dst, zer)


def _onehot(b_ref):
    bvec = b_ref[0, 0, :].reshape(RB, 1)
    return jnp.where(
        bvec == lax.broadcasted_iota(jnp.int32, (RB, G), 1), 1.0, 0.0)


def _split(a):
    ah = a.astype(jnp.bfloat16).astype(jnp.float32)
    return ah, a - ah


def _dotf(x, w):
    return lax.dot_general(x, w, (((1,), (0,)), ((), ())),
                           preferred_element_type=jnp.float32)


def _dot3(x, w):
    xh, xl = _split(x)
    wh, wl = _split(w)
    return _dotf(xh, wh) + (_dotf(xh, wl) + _dotf(xl, wh))


def _dotw(x, w):
    return _dot3(x, w)


def _segdot(oh, h):
    hh, hl = _split(h)
    d = lambda a, b: lax.dot_general(a, b, (((0,), (0,)), ((), ())),
                                     preferred_element_type=jnp.float32)
    return d(oh, hh) + d(oh, hl)


def _vn_gather(oh, vn):
    vh, vl = _split(vn)
    return _dotf(oh, vh) + _dotf(oh, vl)


def _k1_first(x, vn, batch3):
    def body(x_ref, vn_ref, b_ref, hl_ref, seg_ref):
        i = pl.program_id(0)
        oh = _onehot(b_ref)
        hl = x_ref[...] + _vn_gather(oh, vn_ref[...])
        hl_ref[...] = hl
        seg = _segdot(oh, hl)

        @pl.when(i == 0)
        def _():
            seg_ref[...] = seg

        @pl.when(i != 0)
        def _():
            seg_ref[...] += seg

    return pl.pallas_call(
        body,
        grid=(NBLK,),
        in_specs=[
            pl.BlockSpec((RB, D), lambda i: (i, 0)),
            pl.BlockSpec((G, D), lambda i: (0, 0)),
            pl.BlockSpec((1, 1, RB), lambda i: (i, 0, 0)),
        ],
        out_specs=[
            pl.BlockSpec((RB, D), lambda i: (i, 0)),
            pl.BlockSpec((G, D), lambda i: (0, 0)),
        ],
        out_shape=[
            jax.ShapeDtypeStruct((N, D), jnp.float32),
            jax.ShapeDtypeStruct((G, D), jnp.float32),
        ],
    )(x, vn, batch3)


def _rsqrtn(u):
    r = lax.rsqrt(u)
    r = r * (1.5 - 0.5 * u * r * r)
    r = r * (1.5 - 0.5 * u * r * r)
    return r


def _bn_finalize(bm, bM2):
    m = jnp.mean(bm, axis=0, keepdims=True)
    d = bm - m
    var = (jnp.sum(bM2, axis=0, keepdims=True)
           + RB * jnp.sum(d * d, axis=0, keepdims=True)) * (1.0 / N)
    return m, _rsqrtn(var + 1e-5)


def _blk_stats(y):
    mb = jnp.mean(y, axis=0, keepdims=True)
    c = y - mb
    return mb, jnp.sum(c * c, axis=0, keepdims=True)


def _k1_next(z, bm, bM2, gbn, vn, batch3):
    def body(z_ref, bm_ref, bM2_ref, g_ref, vn_ref, b_ref, hl_ref, seg_ref):
        i = pl.program_id(0)
        m, r = _bn_finalize(bm_ref[...], bM2_ref[...])
        h = jnp.maximum((z_ref[...] - m) * r * g_ref[0:1, :] + g_ref[1:2, :],
                        0.0)
        oh = _onehot(b_ref)
        hl = h + _vn_gather(oh, vn_ref[...])
        hl_ref[...] = hl
        seg = _segdot(oh, hl)

        @pl.when(i == 0)
        def _():
            seg_ref[...] = seg

        @pl.when(i != 0)
        def _():
            seg_ref[...] += seg

    return pl.pallas_call(
        body,
        grid=(NBLK,),
        in_specs=[
            pl.BlockSpec((RB, D), lambda i: (i, 0)),
            pl.BlockSpec((NBLK, D), lambda i: (0, 0)),
            pl.BlockSpec((NBLK, D), lambda i: (0, 0)),
            pl.BlockSpec((2, D), lambda i: (0, 0)),
            pl.BlockSpec((G, D), lambda i: (0, 0)),
            pl.BlockSpec((1, 1, RB), lambda i: (i, 0, 0)),
        ],
        out_specs=[
            pl.BlockSpec((RB, D), lambda i: (i, 0)),
            pl.BlockSpec((G, D), lambda i: (0, 0)),
        ],
        out_shape=[
            jax.ShapeDtypeStruct((N, D), jnp.float32),
            jax.ShapeDtypeStruct((G, D), jnp.float32),
        ],
    )(z, bm, bM2, gbn, vn, batch3)


def _k2(hl, agg0, agg1, epsv, w1, b1r):
    def body(hl_ref, a0_ref, a1_ref, e_ref, w_ref, b_ref, y_ref, bm_ref,
             bM2_ref):
        i = pl.program_id(0)
        pre = hl_ref[...] * e_ref[...] + (a0_ref[...] + a1_ref[...])
        y = _dotw(pre, w_ref[...]) + b_ref[...]
        y_ref[...] = y
        mb, M2b = _blk_stats(y)
        bm_ref[pl.ds(i, 1), :] = mb
        bM2_ref[pl.ds(i, 1), :] = M2b

    return pl.pallas_call(
        body,
        grid=(NBLK,),
        in_specs=[
            pl.BlockSpec((RB, D), lambda i: (i, 0)),
            pl.BlockSpec((RB, D), lambda i: (i, 0)),
            pl.BlockSpec((RB, D), lambda i: (i, 0)),
            pl.BlockSpec((1, D), lambda i: (0, 0)),
            pl.BlockSpec((D, HID), lambda i: (0, 0)),
            pl.BlockSpec((1, HID), lambda i: (0, 0)),
        ],
        out_specs=[
            pl.BlockSpec((RB, HID), lambda i: (i, 0)),
            pl.BlockSpec((NBLK, HID), lambda i: (0, 0)),
            pl.BlockSpec((NBLK, HID), lambda i: (0, 0)),
        ],
        out_shape=[
            jax.ShapeDtypeStruct((N, HID), jnp.float32),
            jax.ShapeDtypeStruct((NBLK, HID), jnp.float32),
            jax.ShapeDtypeStruct((NBLK, HID), jnp.float32),
        ],
    )(hl, agg0, agg1, epsv, w1, b1r)


def _k3(y, bm1, bM21, gb1, w2, b2r):
    def body(y_ref, bm_ref, bM2_ref, g_ref, w_ref, b_ref, z_ref, zm_ref,
             zM2_ref):
        i = pl.program_id(0)
        m, r = _bn_finalize(bm_ref[...], bM2_ref[...])
        t = jnp.maximum((y_ref[...] - m) * r * g_ref[0:1, :] + g_ref[1:2, :],
                        0.0)
        z = _dotw(t, w_ref[...]) + b_ref[...]
        z_ref[...] = z
        mb, M2b = _blk_stats(z)
        zm_ref[pl.ds(i, 1), :] = mb
        zM2_ref[pl.ds(i, 1), :] = M2b

    return pl.pallas_call(
        body,
        grid=(NBLK,),
        in_specs=[
            pl.BlockSpec((RB, HID), lambda i: (i, 0)),
            pl.BlockSpec((NBLK, HID), lambda i: (0, 0)),
            pl.BlockSpec((NBLK, HID), lambda i: (0, 0)),
            pl.BlockSpec((2, HID), lambda i: (0, 0)),
            pl.BlockSpec((HID, D), lambda i: (0, 0)),
            pl.BlockSpec((1, D), lambda i: (0, 0)),
        ],
        out_specs=[
            pl.BlockSpec((RB, D), lambda i: (i, 0)),
            pl.BlockSpec((NBLK, D), lambda i: (0, 0)),
            pl.BlockSpec((NBLK, D), lambda i: (0, 0)),
        ],
        out_shape=[
            jax.ShapeDtypeStruct((N, D), jnp.float32),
            jax.ShapeDtypeStruct((NBLK, D), jnp.float32),
            jax.ShapeDtypeStruct((NBLK, D), jnp.float32),
        ],
    )(y, bm1, bM21, gb1, w2, b2r)


def _k5(seg, vn, w1, p1, w2, p2):
    def body(seg_ref, vn_ref, w1_ref, p1_ref, w2_ref, p2_ref, o_ref):
        t0 = seg_ref[...] + vn_ref[...]
        a = _dotw(t0, w1_ref[...]) + p1_ref[0:1, :]
        m = jnp.mean(a, axis=0, keepdims=True)
        ca = a - m
        v = jnp.mean(ca * ca, axis=0, keepdims=True)
        u = jnp.maximum(
            ca * _rsqrtn(v + 1e-5) * p1_ref[1:2, :] + p1_ref[2:3, :],
            0.0)
        z = _dotw(u, w2_ref[...]) + p2_ref[0:1, :]
        m2 = jnp.mean(z, axis=0, keepdims=True)
        cz = z - m2
        v2 = jnp.mean(cz * cz, axis=0, keepdims=True)
        o_ref[...] = jnp.maximum(
            cz * _rsqrtn(v2 + 1e-5) * p2_ref[1:2, :] + p2_ref[2:3, :],
            0.0)

    return pl.pallas_call(
        body,
        out_shape=jax.ShapeDtypeStruct((G, D), jnp.float32),
    )(seg, vn, w1, p1, w2, p2)


def _k6(z, bm, bM2, gbn, batch3, wb, bb):
    def body(z_ref, bm_ref, bM2_ref, g_ref, b_ref, w_ref, bb_ref, o_ref,
             pooled, cnt):
        i = pl.program_id(0)
        m, r = _bn_finalize(bm_ref[...], bM2_ref[...])
        h = (z_ref[...] - m) * r * g_ref[0:1, :] + g_ref[1:2, :]
        oh = _onehot(b_ref)
        pc = _segdot(oh, h)
        cc = _segdot(oh, jnp.ones((RB, G), jnp.float32))

        @pl.when(i == 0)
        def _():
            pooled[...] = pc
            cnt[...] = cc

        @pl.when(i != 0)
        def _():
            pooled[...] += pc
            cnt[...] += cc

        @pl.when(i == NBLK - 1)
        def _():
            c = jnp.maximum(cnt[...], 1.0)
            q = pooled[...] / c
            hg = q + (pooled[...] - q * c) / c
            o_ref[...] = _dot3(hg, w_ref[...]) + bb_ref[...]

    return pl.pallas_call(
        body,
        grid=(NBLK,),
        in_specs=[
            pl.BlockSpec((RB, D), lambda i: (i, 0)),
            pl.BlockSpec((NBLK, D), lambda i: (0, 0)),
            pl.BlockSpec((NBLK, D), lambda i: (0, 0)),
            pl.BlockSpec((2, D), lambda i: (0, 0)),
            pl.BlockSpec((1, 1, RB), lambda i: (i, 0, 0)),
            pl.BlockSpec((D, G), lambda i: (0, 0)),
            pl.BlockSpec((1, G), lambda i: (0, 0)),
        ],
        out_specs=pl.BlockSpec((G, G), lambda i: (0, 0)),
        out_shape=jax.ShapeDtypeStruct((G, G), jnp.float32),
        scratch_shapes=[
            pltpu.VMEM((G, D), jnp.float32),
            pltpu.VMEM((G, G), jnp.float32),
        ],
    )(z, bm, bM2, gbn, batch3, wb, bb)


def kernel(x, edge_attr, gin_eps, gin_W1, gin_b1, gin_g1, gin_be1, gin_W2,
           gin_b2, bn_g, bn_b, vn_emb0, vn_W1, vn_b1, vn_g1, vn_be1, vn_W2,
           vn_b2, vn_g2, vn_be2, mlp_W, mlp_b, edge_index, batch):
    src = edge_index[0]
    dst = edge_index[1]
    batch3 = batch.reshape(NBLK, 1, RB)
    zer = jnp.zeros((CPR, D), jnp.float32)
    vn = jnp.broadcast_to(vn_emb0.reshape(1, D), (G, D))

    hl, seg = _k1_first(x, vn, batch3)
    z = zm = zM2 = None
    for l in range(L):
        aggs = _sc_edge(hl, edge_attr, src, dst, zer)
        epsv = jnp.broadcast_to((1.0 + gin_eps[l]).reshape(1, 1), (1, D))
        epsv = epsv.astype(jnp.float32)
        y, bm1, bM21 = _k2(hl, aggs[0], aggs[1], epsv, gin_W1[l],
                           gin_b1[l].reshape(1, HID))
        gb1 = jnp.stack([gin_g1[l], gin_be1[l]])
        z, zm, zM2 = _k3(y, bm1, bM21, gb1, gin_W2[l], gin_b2[l].reshape(1, D))
        if l < L - 1:
            p1 = jnp.stack([vn_b1[l], vn_g1[l], vn_be1[l]])
            p2 = jnp.stack([vn_b2[l], vn_g2[l], vn_be2[l]])
            vn = _k5(seg, vn, vn_W1[l], p1, vn_W2[l], p2)
            gbn = jnp.stack([bn_g[l], bn_b[l]])
            hl, seg = _k1_next(z, zm, zM2, gbn, vn, batch3)

    gbn4 = jnp.stack([bn_g[L - 1], bn_b[L - 1]])
    wb = jnp.broadcast_to(mlp_W, (D, G))
    bb = jnp.broadcast_to(mlp_b.reshape(1, 1), (1, G))
    out_full = _k6(z, zm, zM2, gbn4, batch3, wb, bb)
    return out_full[:, :NC]
